# Initial kernel scaffold; baseline (speedup 1.0000x reference)
import functools

import jax
import jax.numpy as jnp
from jax import lax
from jax.experimental import pallas as pl
from jax.experimental.pallas import tpu as pltpu
from jax.experimental.pallas import tpu_sc as plsc

N = 10000
E = 320000
H = 128
G = 64
C = 10

NC, NS = 2, 16
NW = NC * NS
EPT = E // NW
KE = 80
NCH = EPT // KE
RPT = N // NS

_mesh = plsc.VectorSubcoreMesh(core_axis_name="c", subcore_axis_name="s")


@functools.partial(
    pl.kernel,
    out_type=jax.ShapeDtypeStruct((2 * N, H), jnp.float32),
    mesh=_mesh,
    scratch_types=[
        pltpu.VMEM((NCH, KE), jnp.int32),
        pltpu.VMEM((NCH, KE), jnp.int32),
        pltpu.VMEM((KE, H), jnp.float32),
        pltpu.VMEM_SHARED((N, H), jnp.float32),
        pltpu.SemaphoreType.DMA,
    ],
)
def _sc_aggregate(x_hbm, src_hbm, dst_hbm, zeros_hbm, out_hbm,
                  src_v, dst_v, rows_v, acc, sem):
    cid = lax.axis_index("c")
    sid = lax.axis_index("s")
    wid = sid * NC + cid
    pltpu.sync_copy(zeros_hbm, acc.at[pl.ds(sid * RPT, RPT)])
    pltpu.sync_copy(src_hbm.at[wid], src_v)
    pltpu.sync_copy(dst_hbm.at[wid], dst_v)
    plsc.subcore_barrier()

    @pl.loop(0, NCH)
    def _chunk(c):
        pltpu.async_copy(x_hbm.at[src_v.at[c]], rows_v, sem).wait()
        pltpu.sync_copy(rows_v, acc.at[dst_v.at[c]], add=True)

    plsc.subcore_barrier()
    pltpu.sync_copy(acc.at[pl.ds(sid * RPT, RPT)],
                    out_hbm.at[pl.ds(cid * N + sid * RPT, RPT)])


def _prep_body(batch_row_ref, batch_col_ref, scale_ref, inv_ref):
    br = batch_row_ref[...]
    bc = batch_col_ref[...]
    gcol = lax.broadcasted_iota(jnp.int32, (G, 1), 0)
    grow = lax.broadcasted_iota(jnp.int32, (1, G), 1)
    onehot = (br == gcol).astype(jnp.float32)
    counts = jnp.sum(onehot, axis=1, keepdims=True)
    inv = jnp.where(counts > 0, 1.0 / counts, 0.0)
    onehot_t = (bc == grow).astype(jnp.float32)
    scale_ref[...] = lax.dot_general(
        onehot_t, inv, (((1,), (0,)), ((), ())),
        preferred_element_type=jnp.float32)
    inv_ref[...] = inv


_prep = pl.pallas_call(
    _prep_body,
    out_shape=(jax.ShapeDtypeStruct((N, 1), jnp.float32),
               jax.ShapeDtypeStruct((G, 1), jnp.float32)),
)

BLK = 1000


def _layer_body(a0_ref, a1_ref, h_ref, scale_ref, wrel_ref, brel_ref,
                wroot_ref, out_ref):
    agg = a0_ref[...] + a1_ref[...]
    out_ref[...] = scale_ref[...] * (
        jnp.dot(agg, wrel_ref[...], preferred_element_type=jnp.float32)
        + brel_ref[...]
        + jnp.dot(h_ref[...], wroot_ref[...],
                  preferred_element_type=jnp.float32))


_layer = pl.pallas_call(
    _layer_body,
    grid=(N // BLK,),
    in_specs=[
        pl.BlockSpec((BLK, H), lambda i: (i, 0)),
        pl.BlockSpec((BLK, H), lambda i: (i + N // BLK, 0)),
        pl.BlockSpec((BLK, H), lambda i: (i, 0)),
        pl.BlockSpec((BLK, 1), lambda i: (i, 0)),
        pl.BlockSpec((H, H), lambda i: (0, 0)),
        pl.BlockSpec((1, H), lambda i: (0, 0)),
        pl.BlockSpec((H, H), lambda i: (0, 0)),
    ],
    out_specs=pl.BlockSpec((BLK, H), lambda i: (i, 0)),
    out_shape=jax.ShapeDtypeStruct((N, H), jnp.float32),
)


def _head_body(batch_row_ref, h_ref, inv_ref, w1_ref, b1_ref, w2_ref,
               b2_ref, out_ref):
    br = batch_row_ref[...]
    gcol = lax.broadcasted_iota(jnp.int32, (G, 1), 0)
    onehot = (br == gcol).astype(jnp.float32)
    pooled = jnp.dot(onehot, h_ref[...],
                     preferred_element_type=jnp.float32) * inv_ref[...]
    z = jnp.maximum(
        jnp.dot(pooled, w1_ref[...], preferred_element_type=jnp.float32)
        + b1_ref[...], 0.0)
    logits = (jnp.dot(z, w2_ref[...], preferred_element_type=jnp.float32)
              + b2_ref[...])
    m = jnp.max(logits, axis=1, keepdims=True)
    lse = jnp.log(jnp.sum(jnp.exp(logits - m), axis=1, keepdims=True)) + m
    out_ref[...] = logits - lse


_head = pl.pallas_call(
    _head_body,
    out_shape=jax.ShapeDtypeStruct((G, C), jnp.float32),
)


def kernel(x, edge_index, batch,
           W_rel1, b_rel1, W_root1,
           W_rel2, b_rel2, W_root2,
           W_rel3, b_rel3, W_root3,
           lin1_W, lin1_b, lin2_W, lin2_b):
    src3 = edge_index[0].reshape(NW, NCH, KE)
    dst3 = edge_index[1].reshape(NW, NCH, KE)
    batch_row = batch.reshape(1, N)
    batch_col = batch.reshape(N, 1)
    zeros = jnp.zeros((RPT, H), jnp.float32)

    scale, inv = _prep(batch_row, batch_col)

    h = x
    for w_rel, b_rel, w_root in ((W_rel1, b_rel1, W_root1),
                                 (W_rel2, b_rel2, W_root2),
                                 (W_rel3, b_rel3, W_root3)):
        agg2 = _sc_aggregate(h, src3, dst3, zeros)
        h = _layer(agg2, agg2, h, scale, w_rel, b_rel.reshape(1, H), w_root)

    return _head(batch_row, h, inv, lin1_W, lin1_b.reshape(1, H),
                 lin2_W, lin2_b.reshape(1, C))


# speedup vs baseline: 6.5959x; 6.5959x over previous
import functools

import jax
import jax.numpy as jnp
from jax import lax
from jax.experimental import pallas as pl
from jax.experimental.pallas import tpu as pltpu
from jax.experimental.pallas import tpu_sc as plsc

N = 10000
E = 320000
H = 128
G = 64
C = 10

NC, NS = 2, 16
NW = NC * NS
EPT = E // NW
KE = 80
NCH = EPT // KE
ZR = 624
ZTAIL = N - NS * ZR

@functools.cache
def _make_sc_aggregate():
    mesh = plsc.VectorSubcoreMesh(core_axis_name="c", subcore_axis_name="s")

    @functools.partial(
        pl.kernel,
        out_type=jax.ShapeDtypeStruct((2 * N, H), jnp.float32),
        mesh=mesh,
        scratch_types=[
            pltpu.VMEM((NCH, KE), jnp.int32),
            pltpu.VMEM((NCH, KE), jnp.int32),
            pltpu.VMEM((KE, H), jnp.float32),
            pltpu.VMEM_SHARED((N, H), jnp.float32),
            pltpu.SemaphoreType.DMA,
        ],
    )
    def _sc_aggregate(x_hbm, src_hbm, dst_hbm, zeros_hbm, out_hbm,
                      src_v, dst_v, rows_v, acc, sem):
        cid = lax.axis_index("c")
        sid = lax.axis_index("s")
        wid = sid * NC + cid
        pltpu.sync_copy(zeros_hbm.at[pl.ds(0, ZR)],
                        acc.at[pl.ds(sid * ZR, ZR)])

        @pl.when(sid == 0)
        def _zero_tail():
            pltpu.sync_copy(zeros_hbm.at[pl.ds(0, ZTAIL)],
                            acc.at[pl.ds(NS * ZR, ZTAIL)])
        pltpu.sync_copy(src_hbm.at[wid], src_v)
        pltpu.sync_copy(dst_hbm.at[wid], dst_v)
        plsc.subcore_barrier()

        @pl.loop(0, NCH)
        def _chunk(c):
            pltpu.async_copy(x_hbm.at[src_v.at[c]], rows_v, sem).wait()
            pltpu.sync_copy(rows_v, acc.at[dst_v.at[c]], add=True)

        plsc.subcore_barrier()
        pltpu.sync_copy(acc.at[pl.ds(sid * ZR, ZR)],
                        out_hbm.at[pl.ds(cid * N + sid * ZR, ZR)])

        @pl.when(sid == 0)
        def _write_tail():
            pltpu.sync_copy(acc.at[pl.ds(NS * ZR, ZTAIL)],
                            out_hbm.at[pl.ds(cid * N + NS * ZR, ZTAIL)])

    return _sc_aggregate


def _prep_body(batch_row_ref, batch_col_ref, scale_ref, inv_ref):
    br = batch_row_ref[...]
    bc = batch_col_ref[...]
    gcol = lax.broadcasted_iota(jnp.int32, (G, 1), 0)
    grow = lax.broadcasted_iota(jnp.int32, (1, G), 1)
    onehot = (br == gcol).astype(jnp.float32)
    counts = jnp.sum(onehot, axis=1, keepdims=True)
    inv = jnp.where(counts > 0, 1.0 / counts, 0.0)
    onehot_t = (bc == grow).astype(jnp.float32)
    scale_ref[...] = lax.dot_general(
        onehot_t, inv, (((1,), (0,)), ((), ())),
        preferred_element_type=jnp.float32)
    inv_ref[...] = inv


_prep = pl.pallas_call(
    _prep_body,
    out_shape=(jax.ShapeDtypeStruct((N, 1), jnp.float32),
               jax.ShapeDtypeStruct((G, 1), jnp.float32)),
)

BLK = 1000


def _layer_body(a0_ref, a1_ref, h_ref, scale_ref, wrel_ref, brel_ref,
                wroot_ref, out_ref):
    agg = a0_ref[...] + a1_ref[...]
    out_ref[...] = scale_ref[...] * (
        jnp.dot(agg, wrel_ref[...], preferred_element_type=jnp.float32)
        + brel_ref[...]
        + jnp.dot(h_ref[...], wroot_ref[...],
                  preferred_element_type=jnp.float32))


_layer = pl.pallas_call(
    _layer_body,
    grid=(N // BLK,),
    in_specs=[
        pl.BlockSpec((BLK, H), lambda i: (i, 0)),
        pl.BlockSpec((BLK, H), lambda i: (i + N // BLK, 0)),
        pl.BlockSpec((BLK, H), lambda i: (i, 0)),
        pl.BlockSpec((BLK, 1), lambda i: (i, 0)),
        pl.BlockSpec((H, H), lambda i: (0, 0)),
        pl.BlockSpec((1, H), lambda i: (0, 0)),
        pl.BlockSpec((H, H), lambda i: (0, 0)),
    ],
    out_specs=pl.BlockSpec((BLK, H), lambda i: (i, 0)),
    out_shape=jax.ShapeDtypeStruct((N, H), jnp.float32),
)


def _head_body(batch_row_ref, h_ref, inv_ref, w1_ref, b1_ref, w2_ref,
               b2_ref, out_ref):
    br = batch_row_ref[...]
    gcol = lax.broadcasted_iota(jnp.int32, (G, 1), 0)
    onehot = (br == gcol).astype(jnp.float32)
    pooled = jnp.dot(onehot, h_ref[...],
                     preferred_element_type=jnp.float32) * inv_ref[...]
    z = jnp.maximum(
        jnp.dot(pooled, w1_ref[...], preferred_element_type=jnp.float32)
        + b1_ref[...], 0.0)
    logits = (jnp.dot(z, w2_ref[...], preferred_element_type=jnp.float32)
              + b2_ref[...])
    m = jnp.max(logits, axis=1, keepdims=True)
    lse = jnp.log(jnp.sum(jnp.exp(logits - m), axis=1, keepdims=True)) + m
    out_ref[...] = logits - lse


_head = pl.pallas_call(
    _head_body,
    out_shape=jax.ShapeDtypeStruct((G, C), jnp.float32),
)


def kernel(x, edge_index, batch,
           W_rel1, b_rel1, W_root1,
           W_rel2, b_rel2, W_root2,
           W_rel3, b_rel3, W_root3,
           lin1_W, lin1_b, lin2_W, lin2_b):
    src3 = edge_index[0].reshape(NW, NCH, KE)
    dst3 = edge_index[1].reshape(NW, NCH, KE)
    batch_row = batch.reshape(1, N)
    batch_col = batch.reshape(N, 1)
    zeros = jnp.zeros((ZR, H), jnp.float32)

    scale, inv = _prep(batch_row, batch_col)

    h = x
    for w_rel, b_rel, w_root in ((W_rel1, b_rel1, W_root1),
                                 (W_rel2, b_rel2, W_root2),
                                 (W_rel3, b_rel3, W_root3)):
        agg2 = _make_sc_aggregate()(h, src3, dst3, zeros)
        h = _layer(agg2, agg2, h, scale, w_rel, b_rel.reshape(1, H), w_root)

    return _head(batch_row, h, inv, lin1_W, lin1_b.reshape(1, H),
                 lin2_W, lin2_b.reshape(1, C))


# speedup vs baseline: 7.2756x; 1.1030x over previous
import functools

import jax
import jax.numpy as jnp
from jax import lax
from jax.experimental import pallas as pl
from jax.experimental.pallas import tpu as pltpu
from jax.experimental.pallas import tpu_sc as plsc

N = 10000
E = 320000
H = 128
G = 64
C = 10

HH = H // 2
NC, NS = 2, 16
NW = NC * NS
EPT = E // NW
KE = 40
NCH = EPT // KE
NBUF = 2
assert NCH % NBUF == 0
ZR = 624
ZTAIL = N - NS * ZR


@functools.cache
def _make_sc_aggregate():
    mesh = plsc.VectorSubcoreMesh(core_axis_name="c", subcore_axis_name="s")

    @functools.partial(
        pl.kernel,
        out_type=jax.ShapeDtypeStruct((2 * N, H), jnp.float32),
        mesh=mesh,
        scratch_types=(
            [pltpu.VMEM((EPT,), jnp.int32),
             pltpu.VMEM((EPT,), jnp.int32),
             pltpu.VMEM_SHARED((N, H), jnp.float32)]
            + [pltpu.VMEM((KE, H), jnp.float32)] * NBUF
            + [pltpu.SemaphoreType.DMA] * (2 * NBUF)
        ),
    )
    def _sc_aggregate(x_hbm, src_hbm, dst_hbm, zeros_hbm, out_hbm,
                      src_v, dst_v, acc, *bufs_and_sems):
        bufs = bufs_and_sems[:NBUF]
        gsems = bufs_and_sems[NBUF:2 * NBUF]
        ssems = bufs_and_sems[2 * NBUF:]
        cid = lax.axis_index("c")
        sid = lax.axis_index("s")
        wid = sid * NC + cid
        pltpu.sync_copy(zeros_hbm.at[pl.ds(0, ZR)],
                        acc.at[pl.ds(sid * ZR, ZR)])

        @pl.when(sid == 0)
        def _zero_tail():
            pltpu.sync_copy(zeros_hbm.at[pl.ds(0, ZTAIL)],
                            acc.at[pl.ds(NS * ZR, ZTAIL)])
        pltpu.sync_copy(src_hbm.at[wid], src_v)
        pltpu.sync_copy(dst_hbm.at[wid], dst_v)
        plsc.subcore_barrier()

        for b in range(NBUF):
            pltpu.async_copy(x_hbm.at[src_v.at[pl.ds(b * KE, KE)]],
                             bufs[b], gsems[b])

        @pl.loop(0, NCH // NBUF)
        def _grp(i):
            for b in range(NBUF):
                c = i * NBUF + b
                pltpu.make_async_copy(
                    x_hbm.at[src_v.at[pl.ds(c * KE, KE)]], bufs[b],
                    gsems[b]).wait()
                pltpu.async_copy(bufs[b],
                                 acc.at[dst_v.at[pl.ds(c * KE, KE)]],
                                 ssems[b], add=True)
            for b in range(NBUF):
                c = i * NBUF + b
                pltpu.make_async_copy(
                    bufs[b], acc.at[dst_v.at[pl.ds(c * KE, KE)]],
                    ssems[b]).wait()

                @pl.when(i + 1 < NCH // NBUF)
                def _next_gather():
                    pltpu.async_copy(
                        x_hbm.at[src_v.at[pl.ds((c + NBUF) * KE, KE)]],
                        bufs[b], gsems[b])

        plsc.subcore_barrier()
        pltpu.sync_copy(acc.at[pl.ds(sid * ZR, ZR)],
                        out_hbm.at[pl.ds(cid * N + sid * ZR, ZR)])

        @pl.when(sid == 0)
        def _write_tail():
            pltpu.sync_copy(acc.at[pl.ds(NS * ZR, ZTAIL)],
                            out_hbm.at[pl.ds(cid * N + NS * ZR, ZTAIL)])

    return _sc_aggregate


def _prep_body(batch_row_ref, batch_col_ref, scale_ref, inv_ref):
    br = batch_row_ref[...]
    bc = batch_col_ref[...]
    gcol = lax.broadcasted_iota(jnp.int32, (G, 1), 0)
    grow = lax.broadcasted_iota(jnp.int32, (1, G), 1)
    onehot = (br == gcol).astype(jnp.float32)
    counts = jnp.sum(onehot, axis=1, keepdims=True)
    inv = jnp.where(counts > 0, 1.0 / counts, 0.0)
    onehot_t = (bc == grow).astype(jnp.float32)
    scale_ref[...] = lax.dot_general(
        onehot_t, inv, (((1,), (0,)), ((), ())),
        preferred_element_type=jnp.float32)
    inv_ref[...] = inv


_prep = pl.pallas_call(
    _prep_body,
    out_shape=(jax.ShapeDtypeStruct((N, 1), jnp.float32),
               jax.ShapeDtypeStruct((G, 1), jnp.float32)),
)

BLK = 1000


def _layer_body(a0_ref, a1_ref, h_ref, scale_ref, wrel_ref, brel_ref,
                wroot_ref, out_ref):
    agg = a0_ref[...] + a1_ref[...]
    out_ref[...] = scale_ref[...] * (
        jnp.dot(agg, wrel_ref[...], preferred_element_type=jnp.float32)
        + brel_ref[...]
        + jnp.dot(h_ref[...], wroot_ref[...],
                  preferred_element_type=jnp.float32))


_layer = pl.pallas_call(
    _layer_body,
    grid=(N // BLK,),
    in_specs=[
        pl.BlockSpec((BLK, H), lambda i: (i, 0)),
        pl.BlockSpec((BLK, H), lambda i: (i + N // BLK, 0)),
        pl.BlockSpec((BLK, H), lambda i: (i, 0)),
        pl.BlockSpec((BLK, 1), lambda i: (i, 0)),
        pl.BlockSpec((H, H), lambda i: (0, 0)),
        pl.BlockSpec((1, H), lambda i: (0, 0)),
        pl.BlockSpec((H, H), lambda i: (0, 0)),
    ],
    out_specs=pl.BlockSpec((BLK, H), lambda i: (i, 0)),
    out_shape=jax.ShapeDtypeStruct((N, H), jnp.float32),
)


def _head_body(batch_row_ref, h_ref, inv_ref, w1_ref, b1_ref, w2_ref,
               b2_ref, out_ref):
    br = batch_row_ref[...]
    gcol = lax.broadcasted_iota(jnp.int32, (G, 1), 0)
    onehot = (br == gcol).astype(jnp.float32)
    pooled = jnp.dot(onehot, h_ref[...],
                     preferred_element_type=jnp.float32) * inv_ref[...]
    z = jnp.maximum(
        jnp.dot(pooled, w1_ref[...], preferred_element_type=jnp.float32)
        + b1_ref[...], 0.0)
    logits = (jnp.dot(z, w2_ref[...], preferred_element_type=jnp.float32)
              + b2_ref[...])
    m = jnp.max(logits, axis=1, keepdims=True)
    lse = jnp.log(jnp.sum(jnp.exp(logits - m), axis=1, keepdims=True)) + m
    out_ref[...] = logits - lse


_head = pl.pallas_call(
    _head_body,
    out_shape=jax.ShapeDtypeStruct((G, C), jnp.float32),
)


def kernel(x, edge_index, batch,
           W_rel1, b_rel1, W_root1,
           W_rel2, b_rel2, W_root2,
           W_rel3, b_rel3, W_root3,
           lin1_W, lin1_b, lin2_W, lin2_b):
    src2 = edge_index[0].reshape(NW, EPT)
    dst2 = edge_index[1].reshape(NW, EPT)
    batch_row = batch.reshape(1, N)
    batch_col = batch.reshape(N, 1)
    zeros = jnp.zeros((ZR, H), jnp.float32)

    scale, inv = _prep(batch_row, batch_col)

    h = x
    for w_rel, b_rel, w_root in ((W_rel1, b_rel1, W_root1),
                                 (W_rel2, b_rel2, W_root2),
                                 (W_rel3, b_rel3, W_root3)):
        agg2 = _make_sc_aggregate()(h, src2, dst2, zeros)
        h = _layer(agg2, agg2, h, scale, w_rel, b_rel.reshape(1, H), w_root)

    return _head(batch_row, h, inv, lin1_W, lin1_b.reshape(1, H),
                 lin2_W, lin2_b.reshape(1, C))


# speedup vs baseline: 10.2780x; 1.4127x over previous
import functools

import jax
import jax.numpy as jnp
from jax import lax
from jax.experimental import pallas as pl
from jax.experimental.pallas import tpu as pltpu
from jax.experimental.pallas import tpu_sc as plsc

N = 10000
E = 320000
H = 128
G = 64
C = 10

HH = H // 2
NC, NS = 2, 16
NW = NC * NS
EPT = E // NW
KE = 40
NCH = EPT // KE
NBUF = 4
NGRP = NCH // NBUF
NEPI = NCH - NGRP * NBUF
ZR = 624
ZTAIL = N - NS * ZR


@functools.cache
def _make_sc_aggregate():
    mesh = plsc.VectorSubcoreMesh(core_axis_name="c", subcore_axis_name="s")

    @functools.partial(
        pl.kernel,
        out_type=jax.ShapeDtypeStruct((2 * N, H), jnp.float32),
        mesh=mesh,
        scratch_types=(
            [pltpu.VMEM((EPT,), jnp.int32),
             pltpu.VMEM((EPT,), jnp.int32),
             pltpu.VMEM_SHARED((N, H), jnp.float32)]
            + [pltpu.VMEM((KE, H), jnp.float32)] * NBUF
            + [pltpu.SemaphoreType.DMA] * (2 * NBUF)
        ),
    )
    def _sc_aggregate(x_hbm, src_hbm, dst_hbm, zeros_hbm, out_hbm,
                      src_v, dst_v, acc, *bufs_and_sems):
        bufs = bufs_and_sems[:NBUF]
        gsems = bufs_and_sems[NBUF:2 * NBUF]
        ssems = bufs_and_sems[2 * NBUF:]
        cid = lax.axis_index("c")
        sid = lax.axis_index("s")
        wid = sid * NC + cid
        pltpu.sync_copy(zeros_hbm.at[pl.ds(0, ZR)],
                        acc.at[pl.ds(sid * ZR, ZR)])

        @pl.when(sid == 0)
        def _zero_tail():
            pltpu.sync_copy(zeros_hbm.at[pl.ds(0, ZTAIL)],
                            acc.at[pl.ds(NS * ZR, ZTAIL)])
        pltpu.sync_copy(src_hbm.at[wid], src_v)
        pltpu.sync_copy(dst_hbm.at[wid], dst_v)
        plsc.subcore_barrier()

        for b in range(NBUF):
            pltpu.async_copy(x_hbm.at[src_v.at[pl.ds(b * KE, KE)]],
                             bufs[b], gsems[b])

        @pl.loop(0, NGRP)
        def _grp(i):
            for b in range(NBUF):
                c = i * NBUF + b
                pltpu.make_async_copy(
                    x_hbm.at[src_v.at[pl.ds(c * KE, KE)]], bufs[b],
                    gsems[b]).wait()
                pltpu.async_copy(bufs[b],
                                 acc.at[dst_v.at[pl.ds(c * KE, KE)]],
                                 ssems[b], add=True)
            for b in range(NBUF):
                c = i * NBUF + b
                pltpu.make_async_copy(
                    bufs[b], acc.at[dst_v.at[pl.ds(c * KE, KE)]],
                    ssems[b]).wait()

                @pl.when(i + 1 < NGRP)
                def _next_gather():
                    pltpu.async_copy(
                        x_hbm.at[src_v.at[pl.ds((c + NBUF) * KE, KE)]],
                        bufs[b], gsems[b])

        @pl.loop(NGRP * NBUF, NCH)
        def _epi(c):
            pltpu.async_copy(x_hbm.at[src_v.at[pl.ds(c * KE, KE)]],
                             bufs[0], gsems[0]).wait()
            pltpu.async_copy(bufs[0], acc.at[dst_v.at[pl.ds(c * KE, KE)]],
                             ssems[0], add=True).wait()

        plsc.subcore_barrier()
        pltpu.sync_copy(acc.at[pl.ds(sid * ZR, ZR)],
                        out_hbm.at[pl.ds(cid * N + sid * ZR, ZR)])

        @pl.when(sid == 0)
        def _write_tail():
            pltpu.sync_copy(acc.at[pl.ds(NS * ZR, ZTAIL)],
                            out_hbm.at[pl.ds(cid * N + NS * ZR, ZTAIL)])

    return _sc_aggregate


def _prep_body(batch_row_ref, batch_col_ref, scale_ref, inv_ref):
    br = batch_row_ref[...]
    bc = batch_col_ref[...]
    gcol = lax.broadcasted_iota(jnp.int32, (G, 1), 0)
    grow = lax.broadcasted_iota(jnp.int32, (1, G), 1)
    onehot = (br == gcol).astype(jnp.float32)
    counts = jnp.sum(onehot, axis=1, keepdims=True)
    inv = jnp.where(counts > 0, 1.0 / counts, 0.0)
    onehot_t = (bc == grow).astype(jnp.float32)
    scale_ref[...] = lax.dot_general(
        onehot_t, inv, (((1,), (0,)), ((), ())),
        preferred_element_type=jnp.float32)
    inv_ref[...] = inv


_prep = pl.pallas_call(
    _prep_body,
    out_shape=(jax.ShapeDtypeStruct((N, 1), jnp.float32),
               jax.ShapeDtypeStruct((G, 1), jnp.float32)),
)

BLK = 1000


def _layer_body(a0_ref, a1_ref, h_ref, scale_ref, wrel_ref, brel_ref,
                wroot_ref, out_ref):
    agg = a0_ref[...] + a1_ref[...]
    out_ref[...] = scale_ref[...] * (
        jnp.dot(agg, wrel_ref[...], preferred_element_type=jnp.float32)
        + brel_ref[...]
        + jnp.dot(h_ref[...], wroot_ref[...],
                  preferred_element_type=jnp.float32))


_layer = pl.pallas_call(
    _layer_body,
    grid=(N // BLK,),
    in_specs=[
        pl.BlockSpec((BLK, H), lambda i: (i, 0)),
        pl.BlockSpec((BLK, H), lambda i: (i + N // BLK, 0)),
        pl.BlockSpec((BLK, H), lambda i: (i, 0)),
        pl.BlockSpec((BLK, 1), lambda i: (i, 0)),
        pl.BlockSpec((H, H), lambda i: (0, 0)),
        pl.BlockSpec((1, H), lambda i: (0, 0)),
        pl.BlockSpec((H, H), lambda i: (0, 0)),
    ],
    out_specs=pl.BlockSpec((BLK, H), lambda i: (i, 0)),
    out_shape=jax.ShapeDtypeStruct((N, H), jnp.float32),
)


def _head_body(batch_row_ref, h_ref, inv_ref, w1_ref, b1_ref, w2_ref,
               b2_ref, out_ref):
    br = batch_row_ref[...]
    gcol = lax.broadcasted_iota(jnp.int32, (G, 1), 0)
    onehot = (br == gcol).astype(jnp.float32)
    pooled = jnp.dot(onehot, h_ref[...],
                     preferred_element_type=jnp.float32) * inv_ref[...]
    z = jnp.maximum(
        jnp.dot(pooled, w1_ref[...], preferred_element_type=jnp.float32)
        + b1_ref[...], 0.0)
    logits = (jnp.dot(z, w2_ref[...], preferred_element_type=jnp.float32)
              + b2_ref[...])
    m = jnp.max(logits, axis=1, keepdims=True)
    lse = jnp.log(jnp.sum(jnp.exp(logits - m), axis=1, keepdims=True)) + m
    out_ref[...] = logits - lse


_head = pl.pallas_call(
    _head_body,
    out_shape=jax.ShapeDtypeStruct((G, C), jnp.float32),
)


def kernel(x, edge_index, batch,
           W_rel1, b_rel1, W_root1,
           W_rel2, b_rel2, W_root2,
           W_rel3, b_rel3, W_root3,
           lin1_W, lin1_b, lin2_W, lin2_b):
    src2 = edge_index[0].reshape(NW, EPT)
    dst2 = edge_index[1].reshape(NW, EPT)
    batch_row = batch.reshape(1, N)
    batch_col = batch.reshape(N, 1)
    zeros = jnp.zeros((ZR, H), jnp.float32)

    scale, inv = _prep(batch_row, batch_col)

    h = x
    for w_rel, b_rel, w_root in ((W_rel1, b_rel1, W_root1),
                                 (W_rel2, b_rel2, W_root2),
                                 (W_rel3, b_rel3, W_root3)):
        agg2 = _make_sc_aggregate()(h, src2, dst2, zeros)
        h = _layer(agg2, agg2, h, scale, w_rel, b_rel.reshape(1, H), w_root)

    return _head(batch_row, h, inv, lin1_W, lin1_b.reshape(1, H),
                 lin2_W, lin2_b.reshape(1, C))


# speedup vs baseline: 10.7123x; 1.0423x over previous
import functools

import jax
import jax.numpy as jnp
from jax import lax
from jax.experimental import pallas as pl
from jax.experimental.pallas import tpu as pltpu
from jax.experimental.pallas import tpu_sc as plsc

N = 10000
E = 320000
H = 128
G = 64
C = 10

HH = H // 2
NC, NS = 2, 16
NW = NC * NS
EPT = E // NW
KE = 40
NCH = EPT // KE
NBUF = 5
NGRP = NCH // NBUF
NEPI = NCH - NGRP * NBUF
ZR = 624
ZTAIL = N - NS * ZR


@functools.cache
def _make_sc_aggregate():
    mesh = plsc.VectorSubcoreMesh(core_axis_name="c", subcore_axis_name="s")

    @functools.partial(
        pl.kernel,
        out_type=jax.ShapeDtypeStruct((2 * N, H), jnp.float32),
        mesh=mesh,
        scratch_types=(
            [pltpu.VMEM((EPT,), jnp.int32),
             pltpu.VMEM((EPT,), jnp.int32),
             pltpu.VMEM_SHARED((N, H), jnp.float32)]
            + [pltpu.VMEM((KE, H), jnp.float32)] * NBUF
            + [pltpu.SemaphoreType.DMA] * (2 * NBUF)
        ),
    )
    def _sc_aggregate(x_hbm, src_hbm, dst_hbm, zeros_hbm, out_hbm,
                      src_v, dst_v, acc, *bufs_and_sems):
        bufs = bufs_and_sems[:NBUF]
        gsems = bufs_and_sems[NBUF:2 * NBUF]
        ssems = bufs_and_sems[2 * NBUF:]
        cid = lax.axis_index("c")
        sid = lax.axis_index("s")
        wid = sid * NC + cid
        pltpu.sync_copy(zeros_hbm.at[pl.ds(0, ZR)],
                        acc.at[pl.ds(sid * ZR, ZR)])

        @pl.when(sid == 0)
        def _zero_tail():
            pltpu.sync_copy(zeros_hbm.at[pl.ds(0, ZTAIL)],
                            acc.at[pl.ds(NS * ZR, ZTAIL)])
        pltpu.sync_copy(src_hbm.at[wid], src_v)
        pltpu.sync_copy(dst_hbm.at[wid], dst_v)
        plsc.subcore_barrier()

        for b in range(NBUF):
            pltpu.async_copy(x_hbm.at[src_v.at[pl.ds(b * KE, KE)]],
                             bufs[b], gsems[b])

        @pl.loop(0, NGRP)
        def _grp(i):
            for b in range(NBUF):
                c = i * NBUF + b
                pltpu.make_async_copy(
                    x_hbm.at[src_v.at[pl.ds(c * KE, KE)]], bufs[b],
                    gsems[b]).wait()
                pltpu.async_copy(bufs[b],
                                 acc.at[dst_v.at[pl.ds(c * KE, KE)]],
                                 ssems[b], add=True)
            for b in range(NBUF):
                c = i * NBUF + b
                pltpu.make_async_copy(
                    bufs[b], acc.at[dst_v.at[pl.ds(c * KE, KE)]],
                    ssems[b]).wait()

                @pl.when(i + 1 < NGRP)
                def _next_gather():
                    pltpu.async_copy(
                        x_hbm.at[src_v.at[pl.ds((c + NBUF) * KE, KE)]],
                        bufs[b], gsems[b])

        @pl.loop(NGRP * NBUF, NCH)
        def _epi(c):
            pltpu.async_copy(x_hbm.at[src_v.at[pl.ds(c * KE, KE)]],
                             bufs[0], gsems[0]).wait()
            pltpu.async_copy(bufs[0], acc.at[dst_v.at[pl.ds(c * KE, KE)]],
                             ssems[0], add=True).wait()

        plsc.subcore_barrier()
        pltpu.sync_copy(acc.at[pl.ds(sid * ZR, ZR)],
                        out_hbm.at[pl.ds(cid * N + sid * ZR, ZR)])

        @pl.when(sid == 0)
        def _write_tail():
            pltpu.sync_copy(acc.at[pl.ds(NS * ZR, ZTAIL)],
                            out_hbm.at[pl.ds(cid * N + NS * ZR, ZTAIL)])

    return _sc_aggregate


def _prep_body(batch_row_ref, batch_col_ref, scale_ref, inv_ref):
    br = batch_row_ref[...]
    bc = batch_col_ref[...]
    gcol = lax.broadcasted_iota(jnp.int32, (G, 1), 0)
    grow = lax.broadcasted_iota(jnp.int32, (1, G), 1)
    onehot = (br == gcol).astype(jnp.float32)
    counts = jnp.sum(onehot, axis=1, keepdims=True)
    inv = jnp.where(counts > 0, 1.0 / counts, 0.0)
    onehot_t = (bc == grow).astype(jnp.float32)
    scale_ref[...] = lax.dot_general(
        onehot_t, inv, (((1,), (0,)), ((), ())),
        preferred_element_type=jnp.float32)
    inv_ref[...] = inv


_prep = pl.pallas_call(
    _prep_body,
    out_shape=(jax.ShapeDtypeStruct((N, 1), jnp.float32),
               jax.ShapeDtypeStruct((G, 1), jnp.float32)),
)

BLK = 1000


def _layer_body(a0_ref, a1_ref, h_ref, scale_ref, wrel_ref, brel_ref,
                wroot_ref, out_ref):
    agg = a0_ref[...] + a1_ref[...]
    out_ref[...] = scale_ref[...] * (
        jnp.dot(agg, wrel_ref[...], preferred_element_type=jnp.float32)
        + brel_ref[...]
        + jnp.dot(h_ref[...], wroot_ref[...],
                  preferred_element_type=jnp.float32))


_layer = pl.pallas_call(
    _layer_body,
    grid=(N // BLK,),
    in_specs=[
        pl.BlockSpec((BLK, H), lambda i: (i, 0)),
        pl.BlockSpec((BLK, H), lambda i: (i + N // BLK, 0)),
        pl.BlockSpec((BLK, H), lambda i: (i, 0)),
        pl.BlockSpec((BLK, 1), lambda i: (i, 0)),
        pl.BlockSpec((H, H), lambda i: (0, 0)),
        pl.BlockSpec((1, H), lambda i: (0, 0)),
        pl.BlockSpec((H, H), lambda i: (0, 0)),
    ],
    out_specs=pl.BlockSpec((BLK, H), lambda i: (i, 0)),
    out_shape=jax.ShapeDtypeStruct((N, H), jnp.float32),
)


def _head_body(batch_row_ref, h_ref, inv_ref, w1_ref, b1_ref, w2_ref,
               b2_ref, out_ref):
    br = batch_row_ref[...]
    gcol = lax.broadcasted_iota(jnp.int32, (G, 1), 0)
    onehot = (br == gcol).astype(jnp.float32)
    pooled = jnp.dot(onehot, h_ref[...],
                     preferred_element_type=jnp.float32) * inv_ref[...]
    z = jnp.maximum(
        jnp.dot(pooled, w1_ref[...], preferred_element_type=jnp.float32)
        + b1_ref[...], 0.0)
    logits = (jnp.dot(z, w2_ref[...], preferred_element_type=jnp.float32)
              + b2_ref[...])
    m = jnp.max(logits, axis=1, keepdims=True)
    lse = jnp.log(jnp.sum(jnp.exp(logits - m), axis=1, keepdims=True)) + m
    out_ref[...] = logits - lse


_head = pl.pallas_call(
    _head_body,
    out_shape=jax.ShapeDtypeStruct((G, C), jnp.float32),
)


def kernel(x, edge_index, batch,
           W_rel1, b_rel1, W_root1,
           W_rel2, b_rel2, W_root2,
           W_rel3, b_rel3, W_root3,
           lin1_W, lin1_b, lin2_W, lin2_b):
    src2 = edge_index[0].reshape(NW, EPT)
    dst2 = edge_index[1].reshape(NW, EPT)
    batch_row = batch.reshape(1, N)
    batch_col = batch.reshape(N, 1)
    zeros = jnp.zeros((ZR, H), jnp.float32)

    scale, inv = _prep(batch_row, batch_col)

    h = x
    for w_rel, b_rel, w_root in ((W_rel1, b_rel1, W_root1),
                                 (W_rel2, b_rel2, W_root2),
                                 (W_rel3, b_rel3, W_root3)):
        agg2 = _make_sc_aggregate()(h, src2, dst2, zeros)
        h = _layer(agg2, agg2, h, scale, w_rel, b_rel.reshape(1, H), w_root)

    return _head(batch_row, h, inv, lin1_W, lin1_b.reshape(1, H),
                 lin2_W, lin2_b.reshape(1, C))
